# baseline (speedup 1.0000x reference)
import functools

import jax
import jax.numpy as jnp
from jax import lax
from jax.experimental import pallas as pl
from jax.experimental.pallas import tpu as pltpu
from jax.experimental.pallas import tpu_sc as plsc

B, S, D = 1, 2048, 768
H = 12
DH = D // H
E = 8
K = 2
DFF = 4 * D

BM = 768
T = (S * K) // BM + E
TPAD = T * BM
EP = 128
RT = 256
NW = 32
CHUNK = S // NW


def _ln_qkv_body(x_ref, g_ref, b_ref, w_ref, bias_ref, q_ref, k_ref, v_ref):
    x = x_ref[...]
    m = jnp.mean(x, axis=-1, keepdims=True)
    v = jnp.mean((x - m) ** 2, axis=-1, keepdims=True)
    h = (x - m) / jnp.sqrt(v + 1e-5) * g_ref[...] + b_ref[...]
    qkv = jnp.dot(h, w_ref[...], preferred_element_type=jnp.float32) + bias_ref[...]
    for out_ref, j in ((q_ref, 0), (k_ref, 1), (v_ref, 2)):
        part = qkv[:, j * D:(j + 1) * D].reshape(RT, H, DH)
        out_ref[...] = jnp.transpose(part, (1, 0, 2)).astype(out_ref.dtype)


def _ln_qkv(xf, g, b, w, bias):
    qkv_spec = pl.BlockSpec((H, RT, DH), lambda i: (0, i, 0))
    qkv_bf16 = jax.ShapeDtypeStruct((H, S, DH), jnp.bfloat16)
    qkv_shape = jax.ShapeDtypeStruct((H, S, DH), jnp.float32)
    return pl.pallas_call(
        _ln_qkv_body,
        grid=(S // RT,),
        in_specs=[
            pl.BlockSpec((RT, D), lambda i: (i, 0)),
            pl.BlockSpec((1, D), lambda i: (0, 0)),
            pl.BlockSpec((1, D), lambda i: (0, 0)),
            pl.BlockSpec((D, 3 * D), lambda i: (0, 0)),
            pl.BlockSpec((1, 3 * D), lambda i: (0, 0)),
        ],
        out_specs=[qkv_spec, qkv_spec, qkv_spec],
        out_shape=[qkv_bf16, qkv_bf16, qkv_shape],
    )(xf, g, b, w, bias)


QB = 256


def _attn_band(q3, k3, v3, band):
    kext = (band + 1) * QB

    pre = band * QB

    def body(q_ref, k_ref, v_ref, o_ref):
        q = q_ref[0] * jnp.bfloat16(1.0 / (DH ** 0.5))
        k = k_ref[0]
        s = lax.dot_general(q, k, (((1,), (1,)), ((), ())),
                            preferred_element_type=jnp.float32)
        sd = s[:, pre:]
        row = lax.broadcasted_iota(jnp.int32, (QB, QB), 0)
        col = lax.broadcasted_iota(jnp.int32, (QB, QB), 1)
        sd = jnp.where(row >= col, sd, -1e9)
        ed_m = jnp.max(sd, axis=-1, keepdims=True)
        if band == 0:
            m = ed_m
            e = jnp.exp(sd - m)
            acc = jnp.dot(e, v_ref[0], preferred_element_type=jnp.float32)
            den = jnp.sum(e, axis=-1, keepdims=True)
        else:
            sp = s[:, :pre]
            m = jnp.maximum(jnp.max(sp, axis=-1, keepdims=True), ed_m)
            ep = jnp.exp(sp - m)
            ed = jnp.exp(sd - m)
            acc = (jnp.dot(ep, v_ref[0, :pre, :],
                           preferred_element_type=jnp.float32)
                   + jnp.dot(ed, v_ref[0, pre:, :],
                             preferred_element_type=jnp.float32))
            den = (jnp.sum(ep, axis=-1, keepdims=True)
                   + jnp.sum(ed, axis=-1, keepdims=True))
        o_ref[0] = acc / den

    return pl.pallas_call(
        body,
        grid=(H,),
        in_specs=[
            pl.BlockSpec((1, QB, DH), lambda h: (h, band, 0)),
            pl.BlockSpec((1, kext, DH), lambda h: (h, 0, 0)),
            pl.BlockSpec((1, kext, DH), lambda h: (h, 0, 0)),
        ],
        out_specs=pl.BlockSpec((1, QB, DH), lambda h: (h, 0, 0)),
        out_shape=jax.ShapeDtypeStruct((H, QB, DH), jnp.float32),
    )(q3, k3, v3)


def _attn(q3, k3, v3):
    return [_attn_band(q3, k3, v3, band) for band in range(S // QB)]


_RPB = QB // RT


def _proj_router_body(x_ref, *refs):
    o_refs = refs[:S // QB]
    wo_ref, bo_ref, g_ref, b_ref, wr_ref, xr_ref, h2_ref, probs_ref = refs[S // QB:]
    i = pl.program_id(0)
    o3h = lax.switch(i // _RPB, [lambda r=r: r[...] for r in o_refs])
    o = jnp.transpose(o3h, (1, 0, 2)).reshape(RT, D)
    xr = x_ref[...] + jnp.dot(o, wo_ref[...],
                              preferred_element_type=jnp.float32) + bo_ref[...]
    xr_ref[...] = xr
    m = jnp.mean(xr, axis=-1, keepdims=True)
    v = jnp.mean((xr - m) ** 2, axis=-1, keepdims=True)
    h2 = (xr - m) / jnp.sqrt(v + 1e-5) * g_ref[...] + b_ref[...]
    h2_ref[...] = h2
    logits = jnp.dot(h2, wr_ref[...], preferred_element_type=jnp.float32)
    lanes = lax.broadcasted_iota(jnp.int32, (RT, EP), 1)
    logits = jnp.where(lanes < E, logits, -1e9)
    mx = jnp.max(logits, axis=-1, keepdims=True)
    ex = jnp.exp(logits - mx)
    probs_ref[...] = ex / jnp.sum(ex, axis=-1, keepdims=True)


def _part_spec(p):
    return pl.BlockSpec(
        (H, RT, DH),
        lambda i, p=p: (0, jnp.clip(i - p * _RPB, 0, _RPB - 1), 0))


def _proj_router(xf, o_parts, wo, bo, g, b, wr_pad):
    return pl.pallas_call(
        _proj_router_body,
        grid=(S // RT,),
        in_specs=[
            pl.BlockSpec((RT, D), lambda i: (i, 0)),
            *[_part_spec(p) for p in range(S // QB)],
            pl.BlockSpec((D, D), lambda i: (0, 0)),
            pl.BlockSpec((1, D), lambda i: (0, 0)),
            pl.BlockSpec((1, D), lambda i: (0, 0)),
            pl.BlockSpec((1, D), lambda i: (0, 0)),
            pl.BlockSpec((D, EP), lambda i: (0, 0)),
        ],
        out_specs=[
            pl.BlockSpec((RT, D), lambda i: (i, 0)),
            pl.BlockSpec((RT, D), lambda i: (i, 0)),
            pl.BlockSpec((RT, EP), lambda i: (i, 0)),
        ],
        out_shape=[
            jax.ShapeDtypeStruct((S, D), jnp.float32),
            jax.ShapeDtypeStruct((S, D), jnp.float32),
            jax.ShapeDtypeStruct((S, EP), jnp.float32),
        ],
    )(xf, *o_parts, wo, bo, g, b, wr_pad)


def _meta_body(p_ref, pos0_ref, pos1_ref, g1_ref, g2_ref, te_ref, nv_ref,
               oh1_s, oh2_s, c0_s, c1_s):
    p = p_ref[...]
    lane = lax.broadcasted_iota(jnp.int32, (S, EP), 1)
    v1 = jnp.max(p, axis=1)
    i1 = jnp.min(jnp.where(p == v1[:, None], lane, EP), axis=1)
    p2 = jnp.where(lane == i1[:, None], -1.0, p)
    v2 = jnp.max(p2, axis=1)
    i2 = jnp.min(jnp.where(p2 == v2[:, None], lane, EP), axis=1)
    den = v1 + v2
    g1_ref[...] = (v1 / den)[:, None] * jnp.ones((1, EP), jnp.float32)
    g2_ref[...] = (v2 / den)[:, None] * jnp.ones((1, EP), jnp.float32)

    oh1 = (lane == i1[:, None]).astype(jnp.float32)
    oh2 = (lane == i2[:, None]).astype(jnp.float32)
    oh1_s[...] = oh1
    oh2_s[...] = oh2

    rows = lax.broadcasted_iota(jnp.int32, (128, 128), 0)
    cols = lax.broadcasted_iota(jnp.int32, (128, 128), 1)
    lstrict = (rows > cols).astype(jnp.float32)

    def body(bi, carry):
        c0, c1 = carry
        blk0 = oh1_s[pl.ds(bi * 128, 128), :]
        blk1 = oh2_s[pl.ds(bi * 128, 128), :]
        c0_s[pl.ds(bi * 128, 128), :] = (
            jnp.dot(lstrict, blk0, preferred_element_type=jnp.float32) + c0)
        c1_s[pl.ds(bi * 128, 128), :] = (
            jnp.dot(lstrict, blk1, preferred_element_type=jnp.float32) + c1)
        return (c0 + jnp.sum(blk0, axis=0, keepdims=True),
                c1 + jnp.sum(blk1, axis=0, keepdims=True))

    tot0, tot1 = lax.fori_loop(
        0, S // 128, body,
        (jnp.zeros((1, EP), jnp.float32), jnp.zeros((1, EP), jnp.float32)))

    gsz = tot0 + tot1
    padded = jnp.ceil(gsz * (1.0 / BM)) * BM
    ustrict = (rows < cols).astype(jnp.float32)
    padoff = jnp.dot(padded, ustrict, preferred_element_type=jnp.float32)
    nv_ref[...] = (jnp.sum(padded, axis=1, keepdims=True) * (1.0 / BM)).astype(jnp.int32)

    rank0 = jnp.sum(c0_s[...] * oh1, axis=1)
    rank1 = jnp.sum(c1_s[...] * oh2, axis=1)
    off0 = jnp.sum(padoff * oh1, axis=1)
    off1 = jnp.sum(padoff * oh2, axis=1)
    t0at = jnp.sum(tot0 * oh2, axis=1)
    pos0_ref[...] = (off0 + rank0).astype(jnp.int32)[None, :]
    pos1_ref[...] = (off1 + t0at + rank1).astype(jnp.int32)[None, :]

    ends = padoff + padded
    tstart = (lax.broadcasted_iota(jnp.int32, (NW, EP), 0) * BM).astype(jnp.float32)
    lane32 = lax.broadcasted_iota(jnp.int32, (NW, EP), 1)
    cnt = jnp.sum(jnp.where((lane32 < E) & (ends <= tstart), 1.0, 0.0), axis=1)
    lane1 = lax.broadcasted_iota(jnp.int32, (1, EP), 1).astype(jnp.float32)
    last_e = jnp.max(jnp.where(padded > 0, lane1, 0.0)).astype(jnp.int32)
    te_ref[...] = jnp.minimum(cnt.astype(jnp.int32), last_e)[None, :]


def _meta(probs_pad):
    return pl.pallas_call(
        _meta_body,
        grid=(1,),
        in_specs=[pl.BlockSpec((S, EP), lambda i: (0, 0))],
        out_specs=[
            pl.BlockSpec((1, S), lambda i: (0, 0)),
            pl.BlockSpec((1, S), lambda i: (0, 0)),
            pl.BlockSpec((S, EP), lambda i: (0, 0)),
            pl.BlockSpec((S, EP), lambda i: (0, 0)),
            pl.BlockSpec((1, NW), lambda i: (0, 0)),
            pl.BlockSpec((1, 1), lambda i: (0, 0)),
        ],
        out_shape=[
            jax.ShapeDtypeStruct((1, S), jnp.int32),
            jax.ShapeDtypeStruct((1, S), jnp.int32),
            jax.ShapeDtypeStruct((S, EP), jnp.float32),
            jax.ShapeDtypeStruct((S, EP), jnp.float32),
            jax.ShapeDtypeStruct((1, NW), jnp.int32),
            jax.ShapeDtypeStruct((1, 1), jnp.int32),
        ],
        scratch_shapes=[
            pltpu.VMEM((S, EP), jnp.float32),
            pltpu.VMEM((S, EP), jnp.float32),
            pltpu.VMEM((S, EP), jnp.float32),
            pltpu.VMEM((S, EP), jnp.float32),
        ],
    )(probs_pad)


@functools.cache
def _make_dispatch():
    mesh = plsc.VectorSubcoreMesh(core_axis_name="c", subcore_axis_name="s")

    @functools.partial(
        pl.kernel,
        mesh=mesh,
        out_type=jax.ShapeDtypeStruct((TPAD, D), jnp.float32),
        scratch_types=[
            pltpu.VMEM((CHUNK, D), jnp.float32),
            pltpu.VMEM((CHUNK,), jnp.int32),
            pltpu.VMEM((CHUNK,), jnp.int32),
            pltpu.SemaphoreType.DMA,
        ],
    )
    def _dispatch(h2_hbm, pos0_hbm, pos1_hbm, out_hbm, rows_v, i0_v, i1_v, sem):
        wid = lax.axis_index("s") * 2 + lax.axis_index("c")
        base = wid * CHUNK
        pltpu.sync_copy(h2_hbm.at[pl.ds(base, CHUNK)], rows_v)
        pltpu.sync_copy(pos0_hbm.at[pl.ds(base, CHUNK)], i0_v)
        pltpu.sync_copy(pos1_hbm.at[pl.ds(base, CHUNK)], i1_v)
        cp0 = pltpu.async_copy(rows_v, out_hbm.at[i0_v], sem)
        cp1 = pltpu.async_copy(rows_v, out_hbm.at[i1_v], sem)
        cp0.wait()
        cp1.wait()

    return _dispatch


def _gmm_body(te_ref, nv_ref, x_ref, w1_ref, b1_ref, w2_ref, b2_ref, y_ref):
    del te_ref
    t = pl.program_id(0)

    @pl.when(t < nv_ref[0])
    def _():
        h = jnp.dot(x_ref[...], w1_ref[0], preferred_element_type=jnp.float32)
        h = jax.nn.gelu(h + b1_ref[0], approximate=True)
        y_ref[...] = (
            jnp.dot(h, w2_ref[0], preferred_element_type=jnp.float32) + b2_ref[0]
        )


def _gmm(te, nv, xs, w1, b1, w2, b2):
    grid_spec = pltpu.PrefetchScalarGridSpec(
        num_scalar_prefetch=2,
        grid=(T,),
        in_specs=[
            pl.BlockSpec((BM, D), lambda t, te, nv: (t, 0)),
            pl.BlockSpec((1, D, DFF), lambda t, te, nv: (te[t], 0, 0)),
            pl.BlockSpec((1, 1, DFF), lambda t, te, nv: (te[t], 0, 0)),
            pl.BlockSpec((1, DFF, D), lambda t, te, nv: (te[t], 0, 0)),
            pl.BlockSpec((1, 1, D), lambda t, te, nv: (te[t], 0, 0)),
        ],
        out_specs=pl.BlockSpec((BM, D), lambda t, te, nv: (t, 0)),
    )
    return pl.pallas_call(
        _gmm_body,
        grid_spec=grid_spec,
        out_shape=jax.ShapeDtypeStruct((TPAD, D), jnp.float32),
    )(te, nv, xs, w1, b1, w2, b2)


@functools.cache
def _make_gather():
    mesh = plsc.VectorSubcoreMesh(core_axis_name="c", subcore_axis_name="s")

    @functools.partial(
        pl.kernel,
        mesh=mesh,
        out_type=[
            jax.ShapeDtypeStruct((S, D), jnp.float32),
            jax.ShapeDtypeStruct((S, D), jnp.float32),
        ],
        scratch_types=[
            pltpu.VMEM((CHUNK,), jnp.int32),
            pltpu.VMEM((CHUNK,), jnp.int32),
            pltpu.VMEM((CHUNK, D), jnp.float32),
            pltpu.VMEM((CHUNK, D), jnp.float32),
            pltpu.SemaphoreType.DMA,
        ],
    )
    def _gather(ys_hbm, pos0_hbm, pos1_hbm, a_hbm, b_hbm, i0_v, i1_v, ra_v, rb_v, sem):
        wid = lax.axis_index("s") * 2 + lax.axis_index("c")
        base = wid * CHUNK
        pltpu.sync_copy(pos0_hbm.at[pl.ds(base, CHUNK)], i0_v)
        pltpu.sync_copy(pos1_hbm.at[pl.ds(base, CHUNK)], i1_v)
        cpa = pltpu.async_copy(ys_hbm.at[i0_v], ra_v, sem)
        cpb = pltpu.async_copy(ys_hbm.at[i1_v], rb_v, sem)
        cpa.wait()
        cpb.wait()
        pltpu.sync_copy(ra_v, a_hbm.at[pl.ds(base, CHUNK)])
        pltpu.sync_copy(rb_v, b_hbm.at[pl.ds(base, CHUNK)])

    return _gather


def _combine_body(xr_ref, a_ref, b_ref, g1_ref, g2_ref, out_ref):
    g1 = jnp.broadcast_to(g1_ref[...][:, 0:1], (RT, D))
    g2 = jnp.broadcast_to(g2_ref[...][:, 0:1], (RT, D))
    out_ref[...] = xr_ref[...] + a_ref[...] * g1 + b_ref[...] * g2


def _combine(xr, a, b, g1b, g2b):
    return pl.pallas_call(
        _combine_body,
        grid=(S // RT,),
        in_specs=[
            pl.BlockSpec((RT, D), lambda i: (i, 0)),
            pl.BlockSpec((RT, D), lambda i: (i, 0)),
            pl.BlockSpec((RT, D), lambda i: (i, 0)),
            pl.BlockSpec((RT, EP), lambda i: (i, 0)),
            pl.BlockSpec((RT, EP), lambda i: (i, 0)),
        ],
        out_specs=pl.BlockSpec((RT, D), lambda i: (i, 0)),
        out_shape=jax.ShapeDtypeStruct((S, D), jnp.float32),
    )(xr, a, b, g1b, g2b)


def kernel(x, ln1_g, ln1_b, Wqkv, bqkv, Wo, bo, ln2_g, ln2_b, Wr, W1, b1, W2, b2):
    xf = x.reshape(S, D)
    q3, k3, v3 = _ln_qkv(xf, ln1_g.reshape(1, D), ln1_b.reshape(1, D), Wqkv,
                         bqkv.reshape(1, 3 * D))
    o = _attn(q3, k3, v3)
    wr_pad = jnp.pad(Wr, ((0, 0), (0, EP - E)))
    xr, h2, probs_pad = _proj_router(xf, o, Wo, bo.reshape(1, D),
                                     ln2_g.reshape(1, D), ln2_b.reshape(1, D),
                                     wr_pad)
    pos0, pos1, g1b, g2b, te, nv = _meta(probs_pad)
    pos0 = pos0.reshape(S)
    pos1 = pos1.reshape(S)
    te = te.reshape(NW)[:T]
    nv = nv.reshape(1)
    xs = _make_dispatch()(h2, pos0, pos1)
    ys = _gmm(te, nv, xs, W1, b1.reshape(E, 1, DFF), W2, b2.reshape(E, 1, D))
    a, b2_rows = _make_gather()(ys, pos0, pos1)
    out = _combine(xr, a, b2_rows, g1b, g2b)
    return out.reshape(B, S, D), probs_pad[:, :E].reshape(B, S, E)


# speedup vs baseline: 1.1484x; 1.1484x over previous
import functools

import jax
import jax.numpy as jnp
from jax import lax
from jax.experimental import pallas as pl
from jax.experimental.pallas import tpu as pltpu
from jax.experimental.pallas import tpu_sc as plsc

B, S, D = 1, 2048, 768
H = 12
DH = D // H
E = 8
K = 2
DFF = 4 * D

BM = 768
T = (S * K) // BM + E
TPAD = T * BM
EP = 128
RT = 256
NW = 32
CHUNK = S // NW


def _ln_qkv_body(x_ref, g_ref, b_ref, w_ref, bias_ref, q_ref, k_ref, v_ref):
    x = x_ref[...]
    m = jnp.mean(x, axis=-1, keepdims=True)
    v = jnp.mean((x - m) ** 2, axis=-1, keepdims=True)
    h = (x - m) / jnp.sqrt(v + 1e-5) * g_ref[...] + b_ref[...]
    qkv = jnp.dot(h, w_ref[...], preferred_element_type=jnp.float32) + bias_ref[...]
    for out_ref, j in ((q_ref, 0), (k_ref, 1), (v_ref, 2)):
        part = qkv[:, j * D:(j + 1) * D].reshape(RT, H, DH)
        out_ref[...] = jnp.transpose(part, (1, 0, 2)).astype(out_ref.dtype)


def _ln_qkv(xf, g, b, w, bias):
    qkv_spec = pl.BlockSpec((H, RT, DH), lambda i: (0, i, 0))
    qkv_bf16 = jax.ShapeDtypeStruct((H, S, DH), jnp.bfloat16)
    qkv_shape = jax.ShapeDtypeStruct((H, S, DH), jnp.float32)
    return pl.pallas_call(
        _ln_qkv_body,
        grid=(S // RT,),
        in_specs=[
            pl.BlockSpec((RT, D), lambda i: (i, 0)),
            pl.BlockSpec((1, D), lambda i: (0, 0)),
            pl.BlockSpec((1, D), lambda i: (0, 0)),
            pl.BlockSpec((D, 3 * D), lambda i: (0, 0)),
            pl.BlockSpec((1, 3 * D), lambda i: (0, 0)),
        ],
        out_specs=[qkv_spec, qkv_spec, qkv_spec],
        out_shape=[qkv_bf16, qkv_bf16, qkv_shape],
    )(xf, g, b, w, bias)


QB = 512


def _attn_band(q3, k3, v3, band):
    kext = (band + 1) * QB

    pre = band * QB

    def body(q_ref, k_ref, v_ref, o_ref):
        q = q_ref[0] * jnp.bfloat16(1.0 / (DH ** 0.5))
        k = k_ref[0]
        s = lax.dot_general(q, k, (((1,), (1,)), ((), ())),
                            preferred_element_type=jnp.float32)
        sd = s[:, pre:]
        row = lax.broadcasted_iota(jnp.int32, (QB, QB), 0)
        col = lax.broadcasted_iota(jnp.int32, (QB, QB), 1)
        sd = jnp.where(row >= col, sd, -1e9)
        ed_m = jnp.max(sd, axis=-1, keepdims=True)
        if band == 0:
            m = ed_m
            e = jnp.exp(sd - m)
            acc = jnp.dot(e, v_ref[0], preferred_element_type=jnp.float32)
            den = jnp.sum(e, axis=-1, keepdims=True)
        else:
            sp = s[:, :pre]
            m = jnp.maximum(jnp.max(sp, axis=-1, keepdims=True), ed_m)
            ep = jnp.exp(sp - m)
            ed = jnp.exp(sd - m)
            acc = (jnp.dot(ep, v_ref[0, :pre, :],
                           preferred_element_type=jnp.float32)
                   + jnp.dot(ed, v_ref[0, pre:, :],
                             preferred_element_type=jnp.float32))
            den = (jnp.sum(ep, axis=-1, keepdims=True)
                   + jnp.sum(ed, axis=-1, keepdims=True))
        o_ref[0] = acc / den

    return pl.pallas_call(
        body,
        grid=(H,),
        in_specs=[
            pl.BlockSpec((1, QB, DH), lambda h: (h, band, 0)),
            pl.BlockSpec((1, kext, DH), lambda h: (h, 0, 0)),
            pl.BlockSpec((1, kext, DH), lambda h: (h, 0, 0)),
        ],
        out_specs=pl.BlockSpec((1, QB, DH), lambda h: (h, 0, 0)),
        out_shape=jax.ShapeDtypeStruct((H, QB, DH), jnp.float32),
    )(q3, k3, v3)


def _attn(q3, k3, v3):
    return [_attn_band(q3, k3, v3, band) for band in range(S // QB)]


_RPB = QB // RT


def _proj_router_body(x_ref, *refs):
    o_refs = refs[:S // QB]
    wo_ref, bo_ref, g_ref, b_ref, wr_ref, xr_ref, h2_ref, probs_ref = refs[S // QB:]
    i = pl.program_id(0)
    o3h = lax.switch(i // _RPB, [lambda r=r: r[...] for r in o_refs])
    o = jnp.transpose(o3h, (1, 0, 2)).reshape(RT, D)
    xr = x_ref[...] + jnp.dot(o, wo_ref[...],
                              preferred_element_type=jnp.float32) + bo_ref[...]
    xr_ref[...] = xr
    m = jnp.mean(xr, axis=-1, keepdims=True)
    v = jnp.mean((xr - m) ** 2, axis=-1, keepdims=True)
    h2 = (xr - m) / jnp.sqrt(v + 1e-5) * g_ref[...] + b_ref[...]
    h2_ref[...] = h2
    logits = jnp.dot(h2, wr_ref[...], preferred_element_type=jnp.float32)
    lanes = lax.broadcasted_iota(jnp.int32, (RT, EP), 1)
    logits = jnp.where(lanes < E, logits, -1e9)
    mx = jnp.max(logits, axis=-1, keepdims=True)
    ex = jnp.exp(logits - mx)
    probs_ref[...] = ex / jnp.sum(ex, axis=-1, keepdims=True)


def _part_spec(p):
    return pl.BlockSpec(
        (H, RT, DH),
        lambda i, p=p: (0, jnp.clip(i - p * _RPB, 0, _RPB - 1), 0))


def _proj_router(xf, o_parts, wo, bo, g, b, wr_pad):
    return pl.pallas_call(
        _proj_router_body,
        grid=(S // RT,),
        in_specs=[
            pl.BlockSpec((RT, D), lambda i: (i, 0)),
            *[_part_spec(p) for p in range(S // QB)],
            pl.BlockSpec((D, D), lambda i: (0, 0)),
            pl.BlockSpec((1, D), lambda i: (0, 0)),
            pl.BlockSpec((1, D), lambda i: (0, 0)),
            pl.BlockSpec((1, D), lambda i: (0, 0)),
            pl.BlockSpec((D, EP), lambda i: (0, 0)),
        ],
        out_specs=[
            pl.BlockSpec((RT, D), lambda i: (i, 0)),
            pl.BlockSpec((RT, D), lambda i: (i, 0)),
            pl.BlockSpec((RT, EP), lambda i: (i, 0)),
        ],
        out_shape=[
            jax.ShapeDtypeStruct((S, D), jnp.float32),
            jax.ShapeDtypeStruct((S, D), jnp.float32),
            jax.ShapeDtypeStruct((S, EP), jnp.float32),
        ],
    )(xf, *o_parts, wo, bo, g, b, wr_pad)


def _meta_body(p_ref, pos0_ref, pos1_ref, g1_ref, g2_ref, te_ref, nv_ref,
               oh1_s, oh2_s, c0_s, c1_s):
    p = p_ref[...]
    lane = lax.broadcasted_iota(jnp.int32, (S, EP), 1)
    v1 = jnp.max(p, axis=1)
    i1 = jnp.min(jnp.where(p == v1[:, None], lane, EP), axis=1)
    p2 = jnp.where(lane == i1[:, None], -1.0, p)
    v2 = jnp.max(p2, axis=1)
    i2 = jnp.min(jnp.where(p2 == v2[:, None], lane, EP), axis=1)
    den = v1 + v2
    g1_ref[...] = (v1 / den)[:, None] * jnp.ones((1, EP), jnp.float32)
    g2_ref[...] = (v2 / den)[:, None] * jnp.ones((1, EP), jnp.float32)

    oh1 = (lane == i1[:, None]).astype(jnp.float32)
    oh2 = (lane == i2[:, None]).astype(jnp.float32)
    oh1_s[...] = oh1
    oh2_s[...] = oh2

    rows = lax.broadcasted_iota(jnp.int32, (128, 128), 0)
    cols = lax.broadcasted_iota(jnp.int32, (128, 128), 1)
    lstrict = (rows > cols).astype(jnp.float32)

    def body(bi, carry):
        c0, c1 = carry
        blk0 = oh1_s[pl.ds(bi * 128, 128), :]
        blk1 = oh2_s[pl.ds(bi * 128, 128), :]
        c0_s[pl.ds(bi * 128, 128), :] = (
            jnp.dot(lstrict, blk0, preferred_element_type=jnp.float32) + c0)
        c1_s[pl.ds(bi * 128, 128), :] = (
            jnp.dot(lstrict, blk1, preferred_element_type=jnp.float32) + c1)
        return (c0 + jnp.sum(blk0, axis=0, keepdims=True),
                c1 + jnp.sum(blk1, axis=0, keepdims=True))

    tot0, tot1 = lax.fori_loop(
        0, S // 128, body,
        (jnp.zeros((1, EP), jnp.float32), jnp.zeros((1, EP), jnp.float32)))

    gsz = tot0 + tot1
    padded = jnp.ceil(gsz * (1.0 / BM)) * BM
    ustrict = (rows < cols).astype(jnp.float32)
    padoff = jnp.dot(padded, ustrict, preferred_element_type=jnp.float32)
    nv_ref[...] = (jnp.sum(padded, axis=1, keepdims=True) * (1.0 / BM)).astype(jnp.int32)

    rank0 = jnp.sum(c0_s[...] * oh1, axis=1)
    rank1 = jnp.sum(c1_s[...] * oh2, axis=1)
    off0 = jnp.sum(padoff * oh1, axis=1)
    off1 = jnp.sum(padoff * oh2, axis=1)
    t0at = jnp.sum(tot0 * oh2, axis=1)
    pos0_ref[...] = (off0 + rank0).astype(jnp.int32)[None, :]
    pos1_ref[...] = (off1 + t0at + rank1).astype(jnp.int32)[None, :]

    ends = padoff + padded
    tstart = (lax.broadcasted_iota(jnp.int32, (NW, EP), 0) * BM).astype(jnp.float32)
    lane32 = lax.broadcasted_iota(jnp.int32, (NW, EP), 1)
    cnt = jnp.sum(jnp.where((lane32 < E) & (ends <= tstart), 1.0, 0.0), axis=1)
    lane1 = lax.broadcasted_iota(jnp.int32, (1, EP), 1).astype(jnp.float32)
    last_e = jnp.max(jnp.where(padded > 0, lane1, 0.0)).astype(jnp.int32)
    te_ref[...] = jnp.minimum(cnt.astype(jnp.int32), last_e)[None, :]


def _meta(probs_pad):
    return pl.pallas_call(
        _meta_body,
        grid=(1,),
        in_specs=[pl.BlockSpec((S, EP), lambda i: (0, 0))],
        out_specs=[
            pl.BlockSpec((1, S), lambda i: (0, 0)),
            pl.BlockSpec((1, S), lambda i: (0, 0)),
            pl.BlockSpec((S, EP), lambda i: (0, 0)),
            pl.BlockSpec((S, EP), lambda i: (0, 0)),
            pl.BlockSpec((1, NW), lambda i: (0, 0)),
            pl.BlockSpec((1, 1), lambda i: (0, 0)),
        ],
        out_shape=[
            jax.ShapeDtypeStruct((1, S), jnp.int32),
            jax.ShapeDtypeStruct((1, S), jnp.int32),
            jax.ShapeDtypeStruct((S, EP), jnp.float32),
            jax.ShapeDtypeStruct((S, EP), jnp.float32),
            jax.ShapeDtypeStruct((1, NW), jnp.int32),
            jax.ShapeDtypeStruct((1, 1), jnp.int32),
        ],
        scratch_shapes=[
            pltpu.VMEM((S, EP), jnp.float32),
            pltpu.VMEM((S, EP), jnp.float32),
            pltpu.VMEM((S, EP), jnp.float32),
            pltpu.VMEM((S, EP), jnp.float32),
        ],
    )(probs_pad)


@functools.cache
def _make_dispatch():
    mesh = plsc.VectorSubcoreMesh(core_axis_name="c", subcore_axis_name="s")

    @functools.partial(
        pl.kernel,
        mesh=mesh,
        out_type=jax.ShapeDtypeStruct((TPAD, D), jnp.float32),
        scratch_types=[
            pltpu.VMEM((CHUNK, D), jnp.float32),
            pltpu.VMEM((CHUNK,), jnp.int32),
            pltpu.VMEM((CHUNK,), jnp.int32),
            pltpu.SemaphoreType.DMA,
        ],
    )
    def _dispatch(h2_hbm, pos0_hbm, pos1_hbm, out_hbm, rows_v, i0_v, i1_v, sem):
        wid = lax.axis_index("s") * 2 + lax.axis_index("c")
        base = wid * CHUNK
        pltpu.sync_copy(h2_hbm.at[pl.ds(base, CHUNK)], rows_v)
        pltpu.sync_copy(pos0_hbm.at[pl.ds(base, CHUNK)], i0_v)
        pltpu.sync_copy(pos1_hbm.at[pl.ds(base, CHUNK)], i1_v)
        cp0 = pltpu.async_copy(rows_v, out_hbm.at[i0_v], sem)
        cp1 = pltpu.async_copy(rows_v, out_hbm.at[i1_v], sem)
        cp0.wait()
        cp1.wait()

    return _dispatch


def _gmm_body(te_ref, nv_ref, x_ref, w1_ref, b1_ref, w2_ref, b2_ref, y_ref):
    del te_ref
    t = pl.program_id(0)

    @pl.when(t < nv_ref[0])
    def _():
        h = jnp.dot(x_ref[...], w1_ref[0], preferred_element_type=jnp.float32)
        h = jax.nn.gelu(h + b1_ref[0], approximate=True)
        y_ref[...] = (
            jnp.dot(h, w2_ref[0], preferred_element_type=jnp.float32) + b2_ref[0]
        )


def _gmm(te, nv, xs, w1, b1, w2, b2):
    grid_spec = pltpu.PrefetchScalarGridSpec(
        num_scalar_prefetch=2,
        grid=(T,),
        in_specs=[
            pl.BlockSpec((BM, D), lambda t, te, nv: (t, 0)),
            pl.BlockSpec((1, D, DFF), lambda t, te, nv: (te[t], 0, 0)),
            pl.BlockSpec((1, 1, DFF), lambda t, te, nv: (te[t], 0, 0)),
            pl.BlockSpec((1, DFF, D), lambda t, te, nv: (te[t], 0, 0)),
            pl.BlockSpec((1, 1, D), lambda t, te, nv: (te[t], 0, 0)),
        ],
        out_specs=pl.BlockSpec((BM, D), lambda t, te, nv: (t, 0)),
    )
    return pl.pallas_call(
        _gmm_body,
        grid_spec=grid_spec,
        out_shape=jax.ShapeDtypeStruct((TPAD, D), jnp.float32),
    )(te, nv, xs, w1, b1, w2, b2)


@functools.cache
def _make_gather():
    mesh = plsc.VectorSubcoreMesh(core_axis_name="c", subcore_axis_name="s")

    @functools.partial(
        pl.kernel,
        mesh=mesh,
        out_type=[
            jax.ShapeDtypeStruct((S, D), jnp.float32),
            jax.ShapeDtypeStruct((S, D), jnp.float32),
        ],
        scratch_types=[
            pltpu.VMEM((CHUNK,), jnp.int32),
            pltpu.VMEM((CHUNK,), jnp.int32),
            pltpu.VMEM((CHUNK, D), jnp.float32),
            pltpu.VMEM((CHUNK, D), jnp.float32),
            pltpu.SemaphoreType.DMA,
        ],
    )
    def _gather(ys_hbm, pos0_hbm, pos1_hbm, a_hbm, b_hbm, i0_v, i1_v, ra_v, rb_v, sem):
        wid = lax.axis_index("s") * 2 + lax.axis_index("c")
        base = wid * CHUNK
        pltpu.sync_copy(pos0_hbm.at[pl.ds(base, CHUNK)], i0_v)
        pltpu.sync_copy(pos1_hbm.at[pl.ds(base, CHUNK)], i1_v)
        cpa = pltpu.async_copy(ys_hbm.at[i0_v], ra_v, sem)
        cpb = pltpu.async_copy(ys_hbm.at[i1_v], rb_v, sem)
        cpa.wait()
        cpb.wait()
        pltpu.sync_copy(ra_v, a_hbm.at[pl.ds(base, CHUNK)])
        pltpu.sync_copy(rb_v, b_hbm.at[pl.ds(base, CHUNK)])

    return _gather


def _combine_body(xr_ref, a_ref, b_ref, g1_ref, g2_ref, out_ref):
    g1 = jnp.broadcast_to(g1_ref[...][:, 0:1], (RT, D))
    g2 = jnp.broadcast_to(g2_ref[...][:, 0:1], (RT, D))
    out_ref[...] = xr_ref[...] + a_ref[...] * g1 + b_ref[...] * g2


def _combine(xr, a, b, g1b, g2b):
    return pl.pallas_call(
        _combine_body,
        grid=(S // RT,),
        in_specs=[
            pl.BlockSpec((RT, D), lambda i: (i, 0)),
            pl.BlockSpec((RT, D), lambda i: (i, 0)),
            pl.BlockSpec((RT, D), lambda i: (i, 0)),
            pl.BlockSpec((RT, EP), lambda i: (i, 0)),
            pl.BlockSpec((RT, EP), lambda i: (i, 0)),
        ],
        out_specs=pl.BlockSpec((RT, D), lambda i: (i, 0)),
        out_shape=jax.ShapeDtypeStruct((S, D), jnp.float32),
    )(xr, a, b, g1b, g2b)


def kernel(x, ln1_g, ln1_b, Wqkv, bqkv, Wo, bo, ln2_g, ln2_b, Wr, W1, b1, W2, b2):
    xf = x.reshape(S, D)
    q3, k3, v3 = _ln_qkv(xf, ln1_g.reshape(1, D), ln1_b.reshape(1, D), Wqkv,
                         bqkv.reshape(1, 3 * D))
    o = _attn(q3, k3, v3)
    wr_pad = jnp.pad(Wr, ((0, 0), (0, EP - E)))
    xr, h2, probs_pad = _proj_router(xf, o, Wo, bo.reshape(1, D),
                                     ln2_g.reshape(1, D), ln2_b.reshape(1, D),
                                     wr_pad)
    pos0, pos1, g1b, g2b, te, nv = _meta(probs_pad)
    pos0 = pos0.reshape(S)
    pos1 = pos1.reshape(S)
    te = te.reshape(NW)[:T]
    nv = nv.reshape(1)
    xs = _make_dispatch()(h2, pos0, pos1)
    ys = _gmm(te, nv, xs, W1, b1.reshape(E, 1, DFF), W2, b2.reshape(E, 1, D))
    a, b2_rows = _make_gather()(ys, pos0, pos1)
    out = _combine(xr, a, b2_rows, g1b, g2b)
    return out.reshape(B, S, D), probs_pad[:, :E].reshape(B, S, E)
